# Initial kernel scaffold; baseline (speedup 1.0000x reference)
import jax
import jax.numpy as jnp
from jax.experimental import pallas as pl


def kernel(cls_logits, pred_boxes, gt_cls, gt_boxes, anchors, scalers):
    raise NotImplementedError("write your pallas kernel here")


# speedup vs baseline: 15.7617x; 15.7617x over previous
import jax
import jax.numpy as jnp
from jax import lax
from jax.experimental import pallas as pl
from jax.experimental.pallas import tpu as pltpu

_TOPK = 10
_EPS = 1e-9
_HALF_PI = 1.5707963267948966
_FOUR_OVER_PI2 = 0.4052847345693511
_NCHUNK = 7


def _atan_ratio(num, den):
    t = jnp.minimum(num, den) / jnp.maximum(num, den)
    s = t * t
    p = jnp.float32(-0.0117212)
    p = p * s + jnp.float32(0.05265332)
    p = p * s + jnp.float32(-0.11643287)
    p = p * s + jnp.float32(0.19354346)
    p = p * s + jnp.float32(-0.33262347)
    p = p * s + jnp.float32(0.99997726)
    f = t * p
    return jnp.where(num > den, _HALF_PI - f, f)


def _body(cls_ref, combo_ref, gtc_ref, gtcT_ref, gtbT_ref,
          cls_out, misc_out, masked_s):
    A = combo_ref.shape[1]
    C = cls_ref.shape[2]
    M = gtbT_ref.shape[2]
    ch = A // _NCHUNK

    gtc = gtc_ref[0]
    onehot = (lax.broadcasted_iota(jnp.int32, (M, C), 1) == gtc
              ).astype(jnp.float32)
    gtb = gtbT_ref[0]
    gx1 = gtb[0:1, :]; gy1 = gtb[1:2, :]; gx2 = gtb[2:3, :]; gy2 = gtb[3:4, :]
    gw = gx2 - gx1; gh = gy2 - gy1
    atg = _atan_ratio(gw, gh + _EPS)
    gsx = gx1 + gx2; gsy = gy1 + gy2

    def pass1(ci, miou):
        sl = pl.ds(ci * ch, ch)
        probs = jax.nn.sigmoid(cls_ref[0, sl, :])
        cls_mat = lax.dot_general(probs, onehot, (((1,), (1,)), ((), ())),
                                  preferred_element_type=jnp.float32,
                                  precision=lax.Precision.HIGHEST)
        cb = combo_ref[0, sl, :]
        px1 = cb[:, 0:1]; py1 = cb[:, 1:2]; px2 = cb[:, 2:3]; py2 = cb[:, 3:4]
        ax = cb[:, 4:5]; ay = cb[:, 5:6]
        inter = (jnp.maximum(jnp.minimum(gx2, px2) - jnp.maximum(gx1, px1), 0.0)
                 * jnp.maximum(jnp.minimum(gy2, py2) - jnp.maximum(gy1, py1),
                               0.0))
        pw = px2 - px1; ph = py2 - py1
        union = gw * gh + pw * ph - inter
        iou = inter / (union + _EPS)
        cw = jnp.maximum(gx2, px2) - jnp.minimum(gx1, px1)
        chh = jnp.maximum(gy2, py2) - jnp.minimum(gy1, py1)
        diag = cw * cw + chh * chh + _EPS
        cdx = gsx - (px1 + px2)
        cdy = gsy - (py1 + py2)
        cdist = 0.25 * (cdx * cdx + cdy * cdy)
        dat = atg - _atan_ratio(pw, ph + _EPS)
        v = _FOUR_OVER_PI2 * dat * dat
        alpha = v / (1.0 - iou + v + _EPS)
        iou_c = jnp.clip(iou - cdist / diag - alpha * v, 0.0, 1.0)
        valid = ((ax >= gx1) & (ax <= gx2) & (ay >= gy1) & (ay <= gy2))
        i2 = iou_c * iou_c
        masked_s[sl, :] = jnp.where(valid, i2 * i2 * i2 * jnp.sqrt(cls_mat),
                                    0.0)
        return jnp.maximum(miou, jnp.max(iou_c, axis=0, keepdims=True))

    max_iou = lax.fori_loop(0, _NCHUNK, pass1, jnp.zeros((1, M), jnp.float32))

    def masked_rowmax(t):
        def step(ci, acc):
            mk = masked_s[pl.ds(ci * ch, ch), :]
            return jnp.maximum(
                acc, jnp.max(jnp.where(mk < t, mk, -1.0), axis=0,
                             keepdims=True))
        return lax.fori_loop(0, _NCHUNK, step,
                             jnp.full((1, M), -1.0, jnp.float32))

    max_target = masked_rowmax(jnp.full((1, M), jnp.inf, jnp.float32))
    t = lax.fori_loop(0, _TOPK - 1, lambda k, tt: masked_rowmax(tt),
                      max_target)

    def pass2(ci, carry):
        sl = pl.ds(ci * ch, ch)
        mk = masked_s[sl, :]
        mask = (mk >= t) & (mk > 0.0)
        tv = jnp.where(mask, mk, 0.0)
        best = jnp.max(tv, axis=1, keepdims=True)
        m_iota = lax.broadcasted_iota(jnp.int32, (ch, M), 1)
        u = jnp.min(jnp.where(tv == best, m_iota, M),
                    axis=1, keepdims=True)
        sel = m_iota == u

        def _take(row):
            return jnp.sum(jnp.where(sel, row, 0.0), axis=1, keepdims=True)

        cb = combo_ref[0, sl, :]
        ax = cb[:, 4:5]; ay = cb[:, 5:6]; scal = cb[:, 6:7]
        valid = ((ax >= gx1) & (ax <= gx2) & (ay >= gy1) & (ay <= gy2))
        valid_any = jnp.any(valid, axis=1, keepdims=True)
        topk_any = jnp.any(mask, axis=1, keepdims=True)
        vmf = (valid_any & topk_any).astype(jnp.float32)

        mt_u = _take(max_target)
        mi_u = _take(max_iou)
        norm = best / (mt_u + _EPS) * mi_u * vmf
        u_cls = _take(gtcT_ref[0]).astype(jnp.int32)
        c_iota = lax.broadcasted_iota(jnp.int32, (ch, C), 1)
        cls_out[0, sl, :] = jnp.where(c_iota == u_cls, norm, 0.0)

        rs = 1.0 / scal
        misc_out[0, sl, :] = jnp.concatenate(
            [_take(gx1) * rs, _take(gy1) * rs, _take(gx2) * rs,
             _take(gy2) * rs, vmf, u.astype(jnp.float32),
             jnp.zeros((ch, 2), jnp.float32)], axis=1)
        return carry

    lax.fori_loop(0, _NCHUNK, pass2, jnp.zeros((), jnp.int32))


def kernel(cls_logits, pred_boxes, gt_cls, gt_boxes, anchors, scalers):
    B, A, C = cls_logits.shape
    M = gt_boxes.shape[1]
    gtc = jnp.clip(gt_cls.astype(jnp.int32), 0)
    gtcT = jnp.transpose(gtc.astype(jnp.float32), (0, 2, 1))
    gtbT = jnp.transpose(gt_boxes, (0, 2, 1))
    aux = jnp.concatenate([anchors, scalers[:, None],
                           jnp.zeros((A, 1), jnp.float32)], axis=1)
    combo = jnp.concatenate(
        [pred_boxes, jnp.broadcast_to(aux[None], (B, A, 4))], axis=2)

    out_shape = [
        jax.ShapeDtypeStruct((B, A, C), jnp.float32),
        jax.ShapeDtypeStruct((B, A, 8), jnp.float32),
    ]
    in_specs = [
        pl.BlockSpec((1, A, C), lambda b: (b, 0, 0)),
        pl.BlockSpec((1, A, 8), lambda b: (b, 0, 0)),
        pl.BlockSpec((1, M, 1), lambda b: (b, 0, 0)),
        pl.BlockSpec((1, 1, M), lambda b: (b, 0, 0)),
        pl.BlockSpec((1, 4, M), lambda b: (b, 0, 0)),
    ]
    out_specs = [
        pl.BlockSpec((1, A, C), lambda b: (b, 0, 0)),
        pl.BlockSpec((1, A, 8), lambda b: (b, 0, 0)),
    ]
    ac, misc = pl.pallas_call(
        _body,
        grid=(B,),
        in_specs=in_specs,
        out_specs=out_specs,
        out_shape=out_shape,
        scratch_shapes=[pltpu.VMEM((A, M), jnp.float32)],
    )(cls_logits, combo, gtc, gtcT, gtbT)
    return (ac, misc[..., 0:4], misc[..., 4],
            misc[..., 5:6].astype(jnp.int32))
